# Initial kernel scaffold; baseline (speedup 1.0000x reference)
import jax
import jax.numpy as jnp
from jax.experimental import pallas as pl


def kernel(x, w1, w_sc, bn_x, bn2, wall, ball, w1e, b1e, w2e, b2e, smat):
    raise NotImplementedError("write your pallas kernel here")


# speedup vs baseline: 1.0895x; 1.0895x over previous
import functools

import jax
import jax.numpy as jnp
from jax import lax
from jax.experimental import pallas as pl
from jax.experimental.pallas import tpu as pltpu

_EPS = 1e-5


def _fold(sum_row, sq_row, gamma, beta, inv_count):
    mean = sum_row * inv_count
    var = sq_row * inv_count - mean * mean
    scale = gamma * lax.rsqrt(var + _EPS)
    shift = beta - mean * scale
    return scale, shift


def _pad2d(h, pad):
    hh, ww, c = h.shape
    z = jnp.zeros((hh, pad, c), h.dtype)
    h = jnp.concatenate([z, h, z], axis=1)
    z = jnp.zeros((pad, ww + 2 * pad, c), h.dtype)
    return jnp.concatenate([z, h, z], axis=0)


def _stats_kernel(x_ref, s_ref, q_ref):
    c = x_ref.shape[-1]
    xv = x_ref[...].reshape(-1, c)
    s_ref[0] = jnp.sum(xv, axis=0, keepdims=True)
    q_ref[0] = jnp.sum(xv * xv, axis=0, keepdims=True)


def _conv1_kernel(x_ref, sum_ref, sq_ref, bnx_ref, w1_ref,
                  c1_ref, s1_ref, s2_ref, *, inv_count, H, W):
    cin = x_ref.shape[-1]
    bnx = bnx_ref[...]
    scale, shift = _fold(sum_ref[...], sq_ref[...], bnx[0:1], bnx[1:2],
                         inv_count)
    h = jnp.maximum(
        x_ref[0] * scale.reshape(1, 1, cin) + shift.reshape(1, 1, cin), 0.0)
    hp = _pad2d(h.astype(jnp.bfloat16), 1)
    acc = None
    for dy in range(3):
        for dx in range(3):
            patch = hp[dy:dy + H, dx:dx + W, :].reshape(H * W, cin)
            d = jnp.dot(patch, w1_ref[dy * 3 + dx],
                        preferred_element_type=jnp.float32)
            acc = d if acc is None else acc + d
    c1_ref[...] = acc.reshape(1, H, W, -1).astype(jnp.bfloat16)
    s1_ref[0] = jnp.sum(acc, axis=0, keepdims=True)
    s2_ref[0] = jnp.sum(acc * acc, axis=0, keepdims=True)


def _psa_kernel(c1_ref, x_ref, s1_ref, s2_ref, sx_ref, qx_ref, bn2_ref,
                bnx_ref, wall_ref, ball_ref, wsc_ref, w1e_ref, b1e_ref,
                w2e_ref, b2e_ref, smat_ref, o_ref, *, inv1, inv2, Ho, Wo):
    C = ball_ref.shape[-1]
    cin = x_ref.shape[-1]
    bn2 = bn2_ref[...]
    scale2, shift2 = _fold(s1_ref[...], s2_ref[...], bn2[0:1], bn2[1:2], inv2)
    h2 = jnp.maximum(
        c1_ref[0] * scale2.reshape(1, 1, C) + shift2.reshape(1, 1, C), 0.0)
    hp = _pad2d(h2.astype(jnp.bfloat16), 4)

    accs = {32: None, 64: None, 96: None, 128: None}
    for dy in range(9):
        for dx in range(9):
            ring = min(dy, dx, 8 - dy, 8 - dx)
            nc = 32 * (min(ring, 3) + 1)
            patch = hp[dy:dy + Ho, dx:dx + Wo, :].reshape(Ho * Wo, C)
            d = jnp.dot(patch, wall_ref[dy * 9 + dx, :, :nc],
                        preferred_element_type=jnp.float32)
            accs[nc] = d if accs[nc] is None else accs[nc] + d
    f128 = accs[128]
    f96 = f128[:, :96] + accs[96]
    f64 = f96[:, :64] + accs[64]
    f32 = f64[:, :32] + accs[32]
    feats = jnp.concatenate(
        [f32, f64[:, 32:64], f96[:, 64:96], f128[:, 96:128]], axis=1)
    feats = feats + ball_ref[...]

    pooled = jnp.sum(feats, axis=0, keepdims=True) * (1.0 / (Ho * Wo))
    z = jnp.maximum(
        jnp.dot(pooled, w1e_ref[...], preferred_element_type=jnp.float32)
        + b1e_ref[...], 0.0)
    logits = (jnp.dot(z, w2e_ref[...], preferred_element_type=jnp.float32)
              + b2e_ref[...])
    se = 1.0 / (1.0 + jnp.exp(-logits))
    e = jnp.exp(se)
    denom = jnp.dot(e, smat_ref[...], preferred_element_type=jnp.float32)
    att = e / denom

    scale_s, shift_s = _fold(sx_ref[...], qx_ref[...], bnx_ref[2:3],
                             bnx_ref[3:4], inv1)
    hs = jnp.maximum(
        x_ref[0] * scale_s.reshape(1, 1, cin) + shift_s.reshape(1, 1, cin),
        0.0)
    sc = jnp.dot(hs.astype(jnp.bfloat16).reshape(Ho * Wo, cin), wsc_ref[...],
                 preferred_element_type=jnp.float32)
    o_ref[...] = (feats * att + sc).reshape(1, Ho, Wo, C)


def kernel(x, w1, w_sc, bn_x, bn2, wall, ball, w1e, b1e, w2e, b2e, smat):
    n, H, W, cin = x.shape
    planes = w1.shape[-1]
    Ho, Wo = H, W
    w1b = w1.astype(jnp.bfloat16)
    wallb = wall.astype(jnp.bfloat16)
    wscb = w_sc.astype(jnp.bfloat16)

    ps, pq = pl.pallas_call(
        _stats_kernel,
        out_shape=(jax.ShapeDtypeStruct((n, 1, cin), jnp.float32),
                   jax.ShapeDtypeStruct((n, 1, cin), jnp.float32)),
        grid=(n,),
        in_specs=[pl.BlockSpec((1, H, W, cin), lambda i: (i, 0, 0, 0))],
        out_specs=(pl.BlockSpec((1, 1, cin), lambda i: (i, 0, 0)),
                   pl.BlockSpec((1, 1, cin), lambda i: (i, 0, 0))),
        compiler_params=pltpu.CompilerParams(
            dimension_semantics=("parallel",)),
    )(x)
    sum_x = jnp.sum(ps, axis=0)
    sq_x = jnp.sum(pq, axis=0)

    conv1 = functools.partial(_conv1_kernel, inv_count=1.0 / (n * H * W),
                              H=H, W=W)
    c1, p1, p2 = pl.pallas_call(
        conv1,
        out_shape=(jax.ShapeDtypeStruct((n, H, W, planes), jnp.bfloat16),
                   jax.ShapeDtypeStruct((n, 1, planes), jnp.float32),
                   jax.ShapeDtypeStruct((n, 1, planes), jnp.float32)),
        grid=(n,),
        in_specs=[
            pl.BlockSpec((1, H, W, cin), lambda i: (i, 0, 0, 0)),
            pl.BlockSpec((1, cin), lambda i: (0, 0)),
            pl.BlockSpec((1, cin), lambda i: (0, 0)),
            pl.BlockSpec(bn_x.shape, lambda i: (0, 0)),
            pl.BlockSpec((9, cin, planes), lambda i: (0, 0, 0)),
        ],
        out_specs=(pl.BlockSpec((1, H, W, planes), lambda i: (i, 0, 0, 0)),
                   pl.BlockSpec((1, 1, planes), lambda i: (i, 0, 0)),
                   pl.BlockSpec((1, 1, planes), lambda i: (i, 0, 0))),
        compiler_params=pltpu.CompilerParams(
            dimension_semantics=("parallel",)),
    )(x, sum_x, sq_x, bn_x, w1b)
    s1 = jnp.sum(p1, axis=0)
    s2 = jnp.sum(p2, axis=0)

    psa = functools.partial(_psa_kernel, inv1=1.0 / (n * H * W),
                            inv2=1.0 / (n * Ho * Wo), Ho=Ho, Wo=Wo)
    out = pl.pallas_call(
        psa,
        out_shape=jax.ShapeDtypeStruct((n, Ho, Wo, planes), jnp.float32),
        grid=(n,),
        in_specs=[
            pl.BlockSpec((1, Ho, Wo, planes), lambda i: (i, 0, 0, 0)),
            pl.BlockSpec((1, H, W, cin), lambda i: (i, 0, 0, 0)),
            pl.BlockSpec((1, planes), lambda i: (0, 0)),
            pl.BlockSpec((1, planes), lambda i: (0, 0)),
            pl.BlockSpec((1, cin), lambda i: (0, 0)),
            pl.BlockSpec((1, cin), lambda i: (0, 0)),
            pl.BlockSpec((2, planes), lambda i: (0, 0)),
            pl.BlockSpec(bn_x.shape, lambda i: (0, 0)),
            pl.BlockSpec((81, planes, planes), lambda i: (0, 0, 0)),
            pl.BlockSpec((1, planes), lambda i: (0, 0)),
            pl.BlockSpec((cin, planes), lambda i: (0, 0)),
            pl.BlockSpec(w1e.shape, lambda i: (0, 0)),
            pl.BlockSpec(b1e.shape, lambda i: (0, 0)),
            pl.BlockSpec(w2e.shape, lambda i: (0, 0)),
            pl.BlockSpec(b2e.shape, lambda i: (0, 0)),
            pl.BlockSpec(smat.shape, lambda i: (0, 0)),
        ],
        out_specs=pl.BlockSpec((1, Ho, Wo, planes), lambda i: (i, 0, 0, 0)),
        compiler_params=pltpu.CompilerParams(
            dimension_semantics=("parallel",)),
    )(c1, x, s1, s2, sum_x, sq_x, bn2, bn_x, wallb, ball, wscb,
      w1e, b1e, w2e, b2e, smat)
    return out
